# baseline (speedup 1.0000x reference)
import functools

import jax
import jax.numpy as jnp
from jax import lax
from jax.experimental import pallas as pl
from jax.experimental.pallas import tpu as pltpu
from jax.experimental.pallas import tpu_sc as plsc

_NUM_PROMPTS = 12
_PROMPT_LEN = 20
_HIDDEN = 4096
_BATCH = 1024

_NC = 2
_NS = 16
_NW = _NC * _NS
_L = 16

_ROWS = _BATCH * _PROMPT_LEN
_RPW = _ROWS // _NW
_CHUNK = 8
_NCHUNKS = _RPW // _CHUNK
_NIDX = _RPW // _L


def _sc_gather(idx, table2d):
    mesh = plsc.VectorSubcoreMesh(core_axis_name="c", subcore_axis_name="s")

    @functools.partial(
        pl.kernel,
        mesh=mesh,
        out_type=jax.ShapeDtypeStruct((_ROWS, _HIDDEN), jnp.float32),
        scratch_types=[
            pltpu.VMEM((_RPW,), jnp.int32),
            pltpu.VMEM((_CHUNK, _HIDDEN), jnp.float32),
            pltpu.VMEM((_CHUNK, _HIDDEN), jnp.float32),
            pltpu.SemaphoreType.DMA,
            pltpu.SemaphoreType.DMA,
            pltpu.SemaphoreType.DMA,
            pltpu.SemaphoreType.DMA,
        ],
    )
    def k(idx_hbm, table_hbm, out_hbm, ridx_v, buf0, buf1,
          g0, g1, s0, s1):
        wid = lax.axis_index("s") * _NC + lax.axis_index("c")
        base = wid * _RPW

        pltpu.sync_copy(idx_hbm.at[pl.ds(base, _RPW)], ridx_v)

        bufs = (buf0, buf1)
        gsems = (g0, g1)
        ssems = (s0, s1)

        def body(i, carry):
            for b in range(2):
                c = i * 2 + b
                row0 = base + c * _CHUNK

                @pl.when(i > 0)
                def _wait_prev_store():
                    pltpu.make_async_copy(
                        bufs[b], out_hbm.at[pl.ds(row0, _CHUNK)], ssems[b]
                    ).wait()

                gather = pltpu.make_async_copy(
                    table_hbm.at[ridx_v.at[pl.ds(c * _CHUNK, _CHUNK)]],
                    bufs[b],
                    gsems[b],
                )
                gather.start()
                gather.wait()
                pltpu.make_async_copy(
                    bufs[b], out_hbm.at[pl.ds(row0, _CHUNK)], ssems[b]
                ).start()
            return carry

        lax.fori_loop(0, _NCHUNKS // 2, body, 0)

        for b in range(2):
            pltpu.make_async_copy(
                bufs[b], out_hbm.at[pl.ds(0, _CHUNK)], ssems[b]
            ).wait()

    return k(idx, table2d)


def kernel(cultural_context, cultural_prompts):
    idx = cultural_context.astype(jnp.int32)
    ridx = (idx[:, None] * _PROMPT_LEN
            + jnp.arange(_PROMPT_LEN, dtype=jnp.int32)[None, :]).reshape(-1)
    table2d = cultural_prompts.reshape(_NUM_PROMPTS * _PROMPT_LEN, _HIDDEN)
    out2d = _sc_gather(ridx, table2d)
    return out2d.reshape(_BATCH, _PROMPT_LEN, _HIDDEN)
